# Initial kernel scaffold; baseline (speedup 1.0000x reference)
import jax
import jax.numpy as jnp
from jax.experimental import pallas as pl


def kernel(x, edge_index, W1, b1, W2, b2, W3, b3):
    raise NotImplementedError("write your pallas kernel here")


# speedup vs baseline: 18.0987x; 18.0987x over previous
import functools

import jax
import jax.numpy as jnp
from jax import lax
from jax.experimental import pallas as pl
from jax.experimental.pallas import tpu as pltpu
from jax.experimental.pallas import tpu_sc as plsc

N = 10000
NPAD = 10240
E = 320000
D = 128
NCORES = 2
NSUB = 16
NTILES = NCORES * NSUB
EPT = E // NTILES
K = 80
NCH = EPT // K
RPT = NPAD // NSUB
BM = 1000


def _sc_mesh():
    return plsc.VectorSubcoreMesh(core_axis_name="c", subcore_axis_name="s")


def _sc_agg(hp, edges3, zero_blk):
    @functools.partial(
        pl.kernel,
        mesh=_sc_mesh(),
        out_type=jax.ShapeDtypeStruct((NCORES, NPAD, D), jnp.float32),
        scratch_types=[
            pltpu.VMEM((2, K), jnp.int32),
            pltpu.VMEM((2, K), jnp.int32),
            pltpu.VMEM((K, D), jnp.float32),
            pltpu.VMEM((K, D), jnp.float32),
            pltpu.VMEM_SHARED((NPAD, D), jnp.float32),
            pltpu.SemaphoreType.DMA,
            pltpu.SemaphoreType.DMA,
        ],
    )
    def k(hp_hbm, edges_hbm, zero_hbm, out_hbm,
          idxa, idxb, bufa, bufb, acc, sema, semb):
        c = lax.axis_index("c")
        s = lax.axis_index("s")
        wid = c * NSUB + s
        row0 = s * RPT

        if True:
            pltpu.sync_copy(zero_hbm, acc.at[pl.ds(row0, RPT)])
            plsc.subcore_barrier()

            pltpu.sync_copy(edges_hbm.at[wid, 0], idxa)
            pltpu.async_copy(hp_hbm.at[idxa.at[0]], bufa, sema)

            def body(j, carry):
                c0 = 2 * j
                c1 = c0 + 1
                pltpu.sync_copy(edges_hbm.at[wid, c1], idxb)
                pltpu.async_copy(hp_hbm.at[idxb.at[0]], bufb, semb)
                pltpu.make_async_copy(hp_hbm.at[idxa.at[0]], bufa, sema).wait()
                pltpu.sync_copy(bufa, acc.at[idxa.at[1]], add=True)
                pltpu.sync_copy(edges_hbm.at[wid, c1 + 1], idxa)
                pltpu.async_copy(hp_hbm.at[idxa.at[0]], bufa, sema)
                pltpu.make_async_copy(hp_hbm.at[idxb.at[0]], bufb, semb).wait()
                pltpu.sync_copy(bufb, acc.at[idxb.at[1]], add=True)
                return carry

            lax.fori_loop(0, (NCH - 1) // 2, body, 0)
            pltpu.make_async_copy(hp_hbm.at[idxa.at[0]], bufa, sema).wait()
            pltpu.sync_copy(bufa, acc.at[idxa.at[1]], add=True)

            plsc.subcore_barrier()
            pltpu.sync_copy(acc.at[pl.ds(row0, RPT)],
                            out_hbm.at[c, pl.ds(row0, RPT)])

    return k(hp, edges3, zero_blk)


def _dinv_block(p):
    deg = p[0, :, 0:1] + p[1, :, 0:1] + 1.0
    return lax.rsqrt(deg)


def _tc_first(x, W, P):
    def body(x_ref, w_ref, p_ref, o_ref):
        dinv = _dinv_block(p_ref[...])
        o_ref[...] = dinv * jnp.dot(x_ref[...], w_ref[...],
                                    preferred_element_type=jnp.float32)

    return pl.pallas_call(
        body,
        grid=(N // BM,),
        in_specs=[
            pl.BlockSpec((BM, D), lambda i: (i, 0)),
            pl.BlockSpec((D, D), lambda i: (0, 0)),
            pl.BlockSpec((NCORES, BM, D), lambda i: (0, i, 0)),
        ],
        out_specs=pl.BlockSpec((BM, D), lambda i: (i, 0)),
        out_shape=jax.ShapeDtypeStruct((N, D), jnp.float32),
    )(x, W, P)


def _tc_mid(A, hp, P, b, W):
    def body(a_ref, hp_ref, p_ref, b_ref, w_ref, o_ref):
        dinv = _dinv_block(p_ref[...])
        a = a_ref[...]
        z = dinv * (a[0] + a[1] + hp_ref[...]) + b_ref[...]
        h = jnp.maximum(z, 0.0)
        o_ref[...] = dinv * jnp.dot(h, w_ref[...],
                                    preferred_element_type=jnp.float32)

    return pl.pallas_call(
        body,
        grid=(N // BM,),
        in_specs=[
            pl.BlockSpec((NCORES, BM, D), lambda i: (0, i, 0)),
            pl.BlockSpec((BM, D), lambda i: (i, 0)),
            pl.BlockSpec((NCORES, BM, D), lambda i: (0, i, 0)),
            pl.BlockSpec((1, D), lambda i: (0, 0)),
            pl.BlockSpec((D, D), lambda i: (0, 0)),
        ],
        out_specs=pl.BlockSpec((BM, D), lambda i: (i, 0)),
        out_shape=jax.ShapeDtypeStruct((N, D), jnp.float32),
    )(A, hp, P, b, W)


def _tc_last(A, hp, P, b):
    def body(a_ref, hp_ref, p_ref, b_ref, o_ref):
        dinv = _dinv_block(p_ref[...])
        a = a_ref[...]
        z = dinv * (a[0] + a[1] + hp_ref[...]) + b_ref[...]
        m = jnp.max(z, axis=1, keepdims=True)
        ez = jnp.exp(z - m)
        o_ref[...] = (z - m) - jnp.log(jnp.sum(ez, axis=1, keepdims=True))

    return pl.pallas_call(
        body,
        grid=(N // BM,),
        in_specs=[
            pl.BlockSpec((NCORES, BM, D), lambda i: (0, i, 0)),
            pl.BlockSpec((BM, D), lambda i: (i, 0)),
            pl.BlockSpec((NCORES, BM, D), lambda i: (0, i, 0)),
            pl.BlockSpec((1, D), lambda i: (0, 0)),
        ],
        out_specs=pl.BlockSpec((BM, D), lambda i: (i, 0)),
        out_shape=jax.ShapeDtypeStruct((N, D), jnp.float32),
    )(A, hp, P, b)


def kernel(x, edge_index, W1, b1, W2, b2, W3, b3):
    edges3 = (edge_index.astype(jnp.int32)
              .reshape(2, NTILES, NCH, K)
              .transpose(1, 2, 0, 3))
    zero_d = jnp.zeros((RPT, D), jnp.float32)
    ones_d = jnp.ones((N, D), jnp.float32)
    b1r = b1.reshape(1, D)
    b2r = b2.reshape(1, D)
    b3r = b3.reshape(1, D)

    P = _sc_agg(ones_d, edges3, zero_d)
    hp1 = _tc_first(x, W1, P)
    A1 = _sc_agg(hp1, edges3, zero_d)
    hp2 = _tc_mid(A1, hp1, P, b1r, W2)
    A2 = _sc_agg(hp2, edges3, zero_d)
    hp3 = _tc_mid(A2, hp2, P, b2r, W3)
    A3 = _sc_agg(hp3, edges3, zero_d)
    return _tc_last(A3, hp3, P, b3r)


# speedup vs baseline: 19.3765x; 1.0706x over previous
import functools

import jax
import jax.numpy as jnp
from jax import lax
from jax.experimental import pallas as pl
from jax.experimental.pallas import tpu as pltpu
from jax.experimental.pallas import tpu_sc as plsc

N = 10000
NPAD = 10240
E = 320000
D = 128
NCORES = 2
NSUB = 16
NTILES = NCORES * NSUB
EPT = E // NTILES
K = 80
NCH = EPT // K
RPT = NPAD // NSUB
BM = 1000


def _sc_mesh():
    return plsc.VectorSubcoreMesh(core_axis_name="c", subcore_axis_name="s")


def _sc_agg(hp, edges3, zero_blk):
    W = hp.shape[1]

    @functools.partial(
        pl.kernel,
        mesh=_sc_mesh(),
        compiler_params=pltpu.CompilerParams(use_tc_tiling_on_sc=(W == D)),
        out_type=jax.ShapeDtypeStruct((NCORES, NPAD, W), jnp.float32),
        scratch_types=[
            pltpu.VMEM((2, K), jnp.int32),
            pltpu.VMEM((2, K), jnp.int32),
            pltpu.VMEM((K, W), jnp.float32),
            pltpu.VMEM((K, W), jnp.float32),
            pltpu.VMEM_SHARED((NPAD, W), jnp.float32),
            pltpu.SemaphoreType.DMA,
            pltpu.SemaphoreType.DMA,
        ],
    )
    def k(hp_hbm, edges_hbm, zero_hbm, out_hbm,
          idxa, idxb, bufa, bufb, acc, sema, semb):
        c = lax.axis_index("c")
        s = lax.axis_index("s")
        wid = c * NSUB + s
        row0 = s * RPT

        if True:
            pltpu.sync_copy(zero_hbm, acc.at[pl.ds(row0, RPT)])
            plsc.subcore_barrier()

            pltpu.sync_copy(edges_hbm.at[wid, 0], idxa)
            pltpu.async_copy(hp_hbm.at[idxa.at[0]], bufa, sema)

            def body(j, carry):
                c0 = 2 * j
                c1 = c0 + 1
                pltpu.sync_copy(edges_hbm.at[wid, c1], idxb)
                pltpu.async_copy(hp_hbm.at[idxb.at[0]], bufb, semb)
                pltpu.make_async_copy(hp_hbm.at[idxa.at[0]], bufa, sema).wait()
                pltpu.sync_copy(bufa, acc.at[idxa.at[1]], add=True)
                pltpu.sync_copy(edges_hbm.at[wid, c1 + 1], idxa)
                pltpu.async_copy(hp_hbm.at[idxa.at[0]], bufa, sema)
                pltpu.make_async_copy(hp_hbm.at[idxb.at[0]], bufb, semb).wait()
                pltpu.sync_copy(bufb, acc.at[idxb.at[1]], add=True)
                return carry

            lax.fori_loop(0, (NCH - 1) // 2, body, 0)
            pltpu.make_async_copy(hp_hbm.at[idxa.at[0]], bufa, sema).wait()
            pltpu.sync_copy(bufa, acc.at[idxa.at[1]], add=True)

            plsc.subcore_barrier()
            pltpu.sync_copy(acc.at[pl.ds(row0, RPT)],
                            out_hbm.at[c, pl.ds(row0, RPT)])

    return k(hp, edges3, zero_blk)


def _dinv_block(p):
    deg = p[0, :, 0:1] + p[1, :, 0:1] + 1.0
    return lax.rsqrt(deg)


def _tc_first(x, W, P):
    def body(x_ref, w_ref, p_ref, o_ref):
        dinv = _dinv_block(p_ref[...])
        o_ref[...] = dinv * jnp.dot(x_ref[...], w_ref[...],
                                    preferred_element_type=jnp.float32)

    return pl.pallas_call(
        body,
        grid=(N // BM,),
        in_specs=[
            pl.BlockSpec((BM, D), lambda i: (i, 0)),
            pl.BlockSpec((D, D), lambda i: (0, 0)),
            pl.BlockSpec((NCORES, BM, 16), lambda i: (0, i, 0)),
        ],
        out_specs=pl.BlockSpec((BM, D), lambda i: (i, 0)),
        out_shape=jax.ShapeDtypeStruct((N, D), jnp.float32),
    )(x, W, P)


def _tc_mid(A, hp, P, b, W):
    def body(a_ref, hp_ref, p_ref, b_ref, w_ref, o_ref):
        dinv = _dinv_block(p_ref[...])
        a = a_ref[...]
        z = dinv * (a[0] + a[1] + hp_ref[...]) + b_ref[...]
        h = jnp.maximum(z, 0.0)
        o_ref[...] = dinv * jnp.dot(h, w_ref[...],
                                    preferred_element_type=jnp.float32)

    return pl.pallas_call(
        body,
        grid=(N // BM,),
        in_specs=[
            pl.BlockSpec((NCORES, BM, D), lambda i: (0, i, 0)),
            pl.BlockSpec((BM, D), lambda i: (i, 0)),
            pl.BlockSpec((NCORES, BM, 16), lambda i: (0, i, 0)),
            pl.BlockSpec((1, D), lambda i: (0, 0)),
            pl.BlockSpec((D, D), lambda i: (0, 0)),
        ],
        out_specs=pl.BlockSpec((BM, D), lambda i: (i, 0)),
        out_shape=jax.ShapeDtypeStruct((N, D), jnp.float32),
    )(A, hp, P, b, W)


def _tc_last(A, hp, P, b):
    def body(a_ref, hp_ref, p_ref, b_ref, o_ref):
        dinv = _dinv_block(p_ref[...])
        a = a_ref[...]
        z = dinv * (a[0] + a[1] + hp_ref[...]) + b_ref[...]
        m = jnp.max(z, axis=1, keepdims=True)
        ez = jnp.exp(z - m)
        o_ref[...] = (z - m) - jnp.log(jnp.sum(ez, axis=1, keepdims=True))

    return pl.pallas_call(
        body,
        grid=(N // BM,),
        in_specs=[
            pl.BlockSpec((NCORES, BM, D), lambda i: (0, i, 0)),
            pl.BlockSpec((BM, D), lambda i: (i, 0)),
            pl.BlockSpec((NCORES, BM, 16), lambda i: (0, i, 0)),
            pl.BlockSpec((1, D), lambda i: (0, 0)),
        ],
        out_specs=pl.BlockSpec((BM, D), lambda i: (i, 0)),
        out_shape=jax.ShapeDtypeStruct((N, D), jnp.float32),
    )(A, hp, P, b)


def kernel(x, edge_index, W1, b1, W2, b2, W3, b3):
    edges3 = (edge_index.astype(jnp.int32)
              .reshape(2, NTILES, NCH, K)
              .transpose(1, 2, 0, 3))
    zero_d = jnp.zeros((RPT, D), jnp.float32)
    zero16 = jnp.zeros((RPT, 16), jnp.float32)
    ones16 = jnp.ones((N, 16), jnp.float32)
    b1r = b1.reshape(1, D)
    b2r = b2.reshape(1, D)
    b3r = b3.reshape(1, D)

    P = _sc_agg(ones16, edges3, zero16)
    hp1 = _tc_first(x, W1, P)
    A1 = _sc_agg(hp1, edges3, zero_d)
    hp2 = _tc_mid(A1, hp1, P, b1r, W2)
    A2 = _sc_agg(hp2, edges3, zero_d)
    hp3 = _tc_mid(A2, hp2, P, b2r, W3)
    A3 = _sc_agg(hp3, edges3, zero_d)
    return _tc_last(A3, hp3, P, b3r)


# speedup vs baseline: 23.3446x; 1.2048x over previous
import functools

import jax
import jax.numpy as jnp
from jax import lax
from jax.experimental import pallas as pl
from jax.experimental.pallas import tpu as pltpu
from jax.experimental.pallas import tpu_sc as plsc

N = 10000
NPAD = 10240
E = 320000
D = 128
NCORES = 2
NSUB = 16
NTILES = NCORES * NSUB
EPT = E // NTILES
K = 125
NCH = EPT // K
RPT = NPAD // NSUB
BM = 1000


def _sc_mesh():
    return plsc.VectorSubcoreMesh(core_axis_name="c", subcore_axis_name="s")


def _sc_agg(hp, edges3, zero_blk):
    W = hp.shape[1]

    @functools.partial(
        pl.kernel,
        mesh=_sc_mesh(),
        compiler_params=pltpu.CompilerParams(use_tc_tiling_on_sc=(W == D)),
        out_type=jax.ShapeDtypeStruct((NCORES, NPAD, W), jnp.float32),
        scratch_types=[
            pltpu.VMEM((2, K), jnp.int32),
            pltpu.VMEM((2, K), jnp.int32),
            pltpu.VMEM((K, W), jnp.float32),
            pltpu.VMEM((K, W), jnp.float32),
            pltpu.VMEM_SHARED((NPAD, W), jnp.float32),
            pltpu.SemaphoreType.DMA,
            pltpu.SemaphoreType.DMA,
        ],
    )
    def k(hp_hbm, edges_hbm, zero_hbm, out_hbm,
          idxa, idxb, bufa, bufb, acc, sema, semb):
        c = lax.axis_index("c")
        s = lax.axis_index("s")
        wid = c * NSUB + s
        row0 = s * RPT

        if True:
            pltpu.sync_copy(zero_hbm, acc.at[pl.ds(row0, RPT)])
            plsc.subcore_barrier()

            pltpu.sync_copy(edges_hbm.at[wid, 0], idxa)
            pltpu.async_copy(hp_hbm.at[idxa.at[0]], bufa, sema)

            def body(j, carry):
                c0 = 2 * j
                c1 = c0 + 1
                pltpu.sync_copy(edges_hbm.at[wid, c1], idxb)
                pltpu.async_copy(hp_hbm.at[idxb.at[0]], bufb, semb)
                pltpu.make_async_copy(hp_hbm.at[idxa.at[0]], bufa, sema).wait()
                pltpu.sync_copy(bufa, acc.at[idxa.at[1]], add=True)
                pltpu.sync_copy(edges_hbm.at[wid, c1 + 1], idxa)
                pltpu.async_copy(hp_hbm.at[idxa.at[0]], bufa, sema)
                pltpu.make_async_copy(hp_hbm.at[idxb.at[0]], bufb, semb).wait()
                pltpu.sync_copy(bufb, acc.at[idxb.at[1]], add=True)
                return carry

            lax.fori_loop(0, (NCH - 1) // 2, body, 0)
            if NCH % 2 == 0:
                pltpu.sync_copy(edges_hbm.at[wid, NCH - 1], idxb)
                pltpu.async_copy(hp_hbm.at[idxb.at[0]], bufb, semb)
                pltpu.make_async_copy(hp_hbm.at[idxa.at[0]], bufa, sema).wait()
                pltpu.sync_copy(bufa, acc.at[idxa.at[1]], add=True)
                pltpu.make_async_copy(hp_hbm.at[idxb.at[0]], bufb, semb).wait()
                pltpu.sync_copy(bufb, acc.at[idxb.at[1]], add=True)
            else:
                pltpu.make_async_copy(hp_hbm.at[idxa.at[0]], bufa, sema).wait()
                pltpu.sync_copy(bufa, acc.at[idxa.at[1]], add=True)

            plsc.subcore_barrier()
            pltpu.sync_copy(acc.at[pl.ds(row0, RPT)],
                            out_hbm.at[c, pl.ds(row0, RPT)])

    return k(hp, edges3, zero_blk)


def _dinv_block(p):
    deg = p[0, :, 0:1] + p[1, :, 0:1] + 1.0
    return lax.rsqrt(deg)


def _tc_first(x, W, P):
    def body(x_ref, w_ref, p_ref, o_ref):
        dinv = _dinv_block(p_ref[...])
        o_ref[...] = dinv * jnp.dot(x_ref[...], w_ref[...],
                                    preferred_element_type=jnp.float32)

    return pl.pallas_call(
        body,
        grid=(N // BM,),
        in_specs=[
            pl.BlockSpec((BM, D), lambda i: (i, 0)),
            pl.BlockSpec((D, D), lambda i: (0, 0)),
            pl.BlockSpec((NCORES, BM, 16), lambda i: (0, i, 0)),
        ],
        out_specs=pl.BlockSpec((BM, D), lambda i: (i, 0)),
        out_shape=jax.ShapeDtypeStruct((N, D), jnp.float32),
    )(x, W, P)


def _tc_mid(A, hp, P, b, W):
    def body(a_ref, hp_ref, p_ref, b_ref, w_ref, o_ref):
        dinv = _dinv_block(p_ref[...])
        a = a_ref[...]
        z = dinv * (a[0] + a[1] + hp_ref[...]) + b_ref[...]
        h = jnp.maximum(z, 0.0)
        o_ref[...] = dinv * jnp.dot(h, w_ref[...],
                                    preferred_element_type=jnp.float32)

    return pl.pallas_call(
        body,
        grid=(N // BM,),
        in_specs=[
            pl.BlockSpec((NCORES, BM, D), lambda i: (0, i, 0)),
            pl.BlockSpec((BM, D), lambda i: (i, 0)),
            pl.BlockSpec((NCORES, BM, 16), lambda i: (0, i, 0)),
            pl.BlockSpec((1, D), lambda i: (0, 0)),
            pl.BlockSpec((D, D), lambda i: (0, 0)),
        ],
        out_specs=pl.BlockSpec((BM, D), lambda i: (i, 0)),
        out_shape=jax.ShapeDtypeStruct((N, D), jnp.float32),
    )(A, hp, P, b, W)


def _tc_last(A, hp, P, b):
    def body(a_ref, hp_ref, p_ref, b_ref, o_ref):
        dinv = _dinv_block(p_ref[...])
        a = a_ref[...]
        z = dinv * (a[0] + a[1] + hp_ref[...]) + b_ref[...]
        m = jnp.max(z, axis=1, keepdims=True)
        ez = jnp.exp(z - m)
        o_ref[...] = (z - m) - jnp.log(jnp.sum(ez, axis=1, keepdims=True))

    return pl.pallas_call(
        body,
        grid=(N // BM,),
        in_specs=[
            pl.BlockSpec((NCORES, BM, D), lambda i: (0, i, 0)),
            pl.BlockSpec((BM, D), lambda i: (i, 0)),
            pl.BlockSpec((NCORES, BM, 16), lambda i: (0, i, 0)),
            pl.BlockSpec((1, D), lambda i: (0, 0)),
        ],
        out_specs=pl.BlockSpec((BM, D), lambda i: (i, 0)),
        out_shape=jax.ShapeDtypeStruct((N, D), jnp.float32),
    )(A, hp, P, b)


def kernel(x, edge_index, W1, b1, W2, b2, W3, b3):
    edges3 = (edge_index.astype(jnp.int32)
              .reshape(2, NTILES, NCH, K)
              .transpose(1, 2, 0, 3))
    zero_d = jnp.zeros((RPT, D), jnp.float32)
    zero16 = jnp.zeros((RPT, 16), jnp.float32)
    ones16 = jnp.ones((N, 16), jnp.float32)
    b1r = b1.reshape(1, D)
    b2r = b2.reshape(1, D)
    b3r = b3.reshape(1, D)

    P = _sc_agg(ones16, edges3, zero16)
    hp1 = _tc_first(x, W1, P)
    A1 = _sc_agg(hp1, edges3, zero_d)
    hp2 = _tc_mid(A1, hp1, P, b1r, W2)
    A2 = _sc_agg(hp2, edges3, zero_d)
    hp3 = _tc_mid(A2, hp2, P, b2r, W3)
    A3 = _sc_agg(hp3, edges3, zero_d)
    return _tc_last(A3, hp3, P, b3r)


# speedup vs baseline: 26.6898x; 1.1433x over previous
import functools

import jax
import jax.numpy as jnp
from jax import lax
from jax.experimental import pallas as pl
from jax.experimental.pallas import tpu as pltpu
from jax.experimental.pallas import tpu_sc as plsc

N = 10000
NPAD = 10240
E = 320000
D = 128
NCORES = 2
NSUB = 16
NTILES = NCORES * NSUB
EPT = E // NTILES
K = 125
NCH = EPT // K
RPT = NPAD // NSUB
BM = 1000


def _sc_mesh():
    return plsc.VectorSubcoreMesh(core_axis_name="c", subcore_axis_name="s")


def _sc_agg(hp, edges3, zero_blk):
    W = hp.shape[1]

    @functools.partial(
        pl.kernel,
        mesh=_sc_mesh(),
        compiler_params=pltpu.CompilerParams(use_tc_tiling_on_sc=(W == D)),
        out_type=jax.ShapeDtypeStruct((NCORES, NPAD, W), jnp.float32),
        scratch_types=[
            pltpu.VMEM((2, K), jnp.int32),
            pltpu.VMEM((2, K), jnp.int32),
            pltpu.VMEM((2, K), jnp.int32),
            pltpu.VMEM((2, K), jnp.int32),
            pltpu.VMEM((K, W), jnp.float32),
            pltpu.VMEM((K, W), jnp.float32),
            pltpu.VMEM_SHARED((NPAD, W), jnp.float32),
            pltpu.SemaphoreType.DMA,
            pltpu.SemaphoreType.DMA,
            pltpu.SemaphoreType.DMA,
            pltpu.SemaphoreType.DMA,
            pltpu.SemaphoreType.DMA,
            pltpu.SemaphoreType.DMA,
        ],
    )
    def k(hp_hbm, edges_hbm, zero_hbm, out_hbm,
          i0, i1, i2, i3, bufa, bufb, acc,
          sema, semb, semi0, semi1, semi2, semi3):
        c = lax.axis_index("c")
        s = lax.axis_index("s")
        wid = c * NSUB + s
        row0 = s * RPT
        ibufs = (i0, i1, i2, i3)
        isems = (semi0, semi1, semi2, semi3)
        rbufs = (bufa, bufb)
        rsems = (sema, semb)

        def idx_load(ch, p):
            pltpu.async_copy(edges_hbm.at[wid, ch], ibufs[p], isems[p])

        def idx_wait(p):
            pltpu.make_async_copy(edges_hbm.at[wid, 0], ibufs[p],
                                  isems[p]).wait()

        def gather(p4, p2):
            pltpu.async_copy(hp_hbm.at[ibufs[p4].at[0]], rbufs[p2], rsems[p2])

        def gather_wait(p2):
            pltpu.make_async_copy(hp_hbm.at[ibufs[0].at[0]], rbufs[p2],
                                  rsems[p2]).wait()

        def scatter(p4, p2):
            pltpu.sync_copy(rbufs[p2], acc.at[ibufs[p4].at[1]], add=True)

        pltpu.sync_copy(zero_hbm, acc.at[pl.ds(row0, RPT)])
        plsc.subcore_barrier()

        pltpu.sync_copy(edges_hbm.at[wid, 0], i0)
        pltpu.sync_copy(edges_hbm.at[wid, 1], i1)
        gather(0, 0)
        gather(1, 1)
        idx_load(2, 2)
        idx_load(3, 3)

        def body(m, carry):
            q = 4 * m
            for p in range(4):
                p2 = p & 1
                gather_wait(p2)
                scatter(p, p2)
                idx_load(q + p + 4, p)
                idx_wait((p + 2) & 3)
                gather((p + 2) & 3, p2)
            return carry

        lax.fori_loop(0, NCH // 4 - 1, body, 0)
        gather_wait(0)
        scatter(0, 0)
        idx_wait(2)
        gather(2, 0)
        gather_wait(1)
        scatter(1, 1)
        idx_wait(3)
        gather(3, 1)
        gather_wait(0)
        scatter(2, 0)
        gather_wait(1)
        scatter(3, 1)

        plsc.subcore_barrier()
        pltpu.sync_copy(acc.at[pl.ds(row0, RPT)],
                        out_hbm.at[c, pl.ds(row0, RPT)])

    return k(hp, edges3, zero_blk)


def _dinv_block(p):
    deg = p[0, :, 0:1] + p[1, :, 0:1] + 1.0
    return lax.rsqrt(deg)


def _tc_first(x, W, P):
    def body(x_ref, w_ref, p_ref, o_ref):
        dinv = _dinv_block(p_ref[...])
        o_ref[...] = dinv * jnp.dot(x_ref[...], w_ref[...],
                                    preferred_element_type=jnp.float32)

    return pl.pallas_call(
        body,
        grid=(N // BM,),
        in_specs=[
            pl.BlockSpec((BM, D), lambda i: (i, 0)),
            pl.BlockSpec((D, D), lambda i: (0, 0)),
            pl.BlockSpec((NCORES, BM, 16), lambda i: (0, i, 0)),
        ],
        out_specs=pl.BlockSpec((BM, D), lambda i: (i, 0)),
        out_shape=jax.ShapeDtypeStruct((N, D), jnp.float32),
    )(x, W, P)


def _tc_mid(A, hp, P, b, W):
    def body(a_ref, hp_ref, p_ref, b_ref, w_ref, o_ref):
        dinv = _dinv_block(p_ref[...])
        a = a_ref[...]
        z = dinv * (a[0] + a[1] + hp_ref[...]) + b_ref[...]
        h = jnp.maximum(z, 0.0)
        o_ref[...] = dinv * jnp.dot(h, w_ref[...],
                                    preferred_element_type=jnp.float32)

    return pl.pallas_call(
        body,
        grid=(N // BM,),
        in_specs=[
            pl.BlockSpec((NCORES, BM, D), lambda i: (0, i, 0)),
            pl.BlockSpec((BM, D), lambda i: (i, 0)),
            pl.BlockSpec((NCORES, BM, 16), lambda i: (0, i, 0)),
            pl.BlockSpec((1, D), lambda i: (0, 0)),
            pl.BlockSpec((D, D), lambda i: (0, 0)),
        ],
        out_specs=pl.BlockSpec((BM, D), lambda i: (i, 0)),
        out_shape=jax.ShapeDtypeStruct((N, D), jnp.float32),
    )(A, hp, P, b, W)


def _tc_last(A, hp, P, b):
    def body(a_ref, hp_ref, p_ref, b_ref, o_ref):
        dinv = _dinv_block(p_ref[...])
        a = a_ref[...]
        z = dinv * (a[0] + a[1] + hp_ref[...]) + b_ref[...]
        m = jnp.max(z, axis=1, keepdims=True)
        ez = jnp.exp(z - m)
        o_ref[...] = (z - m) - jnp.log(jnp.sum(ez, axis=1, keepdims=True))

    return pl.pallas_call(
        body,
        grid=(N // BM,),
        in_specs=[
            pl.BlockSpec((NCORES, BM, D), lambda i: (0, i, 0)),
            pl.BlockSpec((BM, D), lambda i: (i, 0)),
            pl.BlockSpec((NCORES, BM, 16), lambda i: (0, i, 0)),
            pl.BlockSpec((1, D), lambda i: (0, 0)),
        ],
        out_specs=pl.BlockSpec((BM, D), lambda i: (i, 0)),
        out_shape=jax.ShapeDtypeStruct((N, D), jnp.float32),
    )(A, hp, P, b)


def kernel(x, edge_index, W1, b1, W2, b2, W3, b3):
    edges3 = (edge_index.astype(jnp.int32)
              .reshape(2, NTILES, NCH, K)
              .transpose(1, 2, 0, 3))
    zero_d = jnp.zeros((RPT, D), jnp.float32)
    zero16 = jnp.zeros((RPT, 16), jnp.float32)
    ones16 = jnp.ones((N, 16), jnp.float32)
    b1r = b1.reshape(1, D)
    b2r = b2.reshape(1, D)
    b3r = b3.reshape(1, D)

    P = _sc_agg(ones16, edges3, zero16)
    hp1 = _tc_first(x, W1, P)
    A1 = _sc_agg(hp1, edges3, zero_d)
    hp2 = _tc_mid(A1, hp1, P, b1r, W2)
    A2 = _sc_agg(hp2, edges3, zero_d)
    hp3 = _tc_mid(A2, hp2, P, b2r, W3)
    A3 = _sc_agg(hp3, edges3, zero_d)
    return _tc_last(A3, hp3, P, b3r)


# speedup vs baseline: 27.7787x; 1.0408x over previous
import functools

import jax
import jax.numpy as jnp
from jax import lax
from jax.experimental import pallas as pl
from jax.experimental.pallas import tpu as pltpu
from jax.experimental.pallas import tpu_sc as plsc

N = 10000
NPAD = 10240
E = 320000
D = 128
NCORES = 2
NSUB = 16
NTILES = NCORES * NSUB
EPT = E // NTILES
K = 125
NCH = EPT // K
RPT = NPAD // NSUB
BM = 1000


def _sc_mesh():
    return plsc.VectorSubcoreMesh(core_axis_name="c", subcore_axis_name="s")


def _sc_agg(hp, edges3, zero_blk):
    W = hp.shape[1]
    NCH_, K_ = edges3.shape[1], edges3.shape[3]

    @functools.partial(
        pl.kernel,
        mesh=_sc_mesh(),
        compiler_params=pltpu.CompilerParams(use_tc_tiling_on_sc=(W == D)),
        out_type=jax.ShapeDtypeStruct((NCORES, NPAD, W), jnp.float32),
        scratch_types=[
            pltpu.VMEM((2, K_), jnp.int32),
            pltpu.VMEM((2, K_), jnp.int32),
            pltpu.VMEM((2, K_), jnp.int32),
            pltpu.VMEM((2, K_), jnp.int32),
            pltpu.VMEM((K_, W), jnp.float32),
            pltpu.VMEM((K_, W), jnp.float32),
            pltpu.VMEM_SHARED((NPAD, W), jnp.float32),
            pltpu.SemaphoreType.DMA,
            pltpu.SemaphoreType.DMA,
            pltpu.SemaphoreType.DMA,
            pltpu.SemaphoreType.DMA,
            pltpu.SemaphoreType.DMA,
            pltpu.SemaphoreType.DMA,
        ],
    )
    def k(hp_hbm, edges_hbm, zero_hbm, out_hbm,
          i0, i1, i2, i3, bufa, bufb, acc,
          sema, semb, semi0, semi1, semi2, semi3):
        c = lax.axis_index("c")
        s = lax.axis_index("s")
        wid = c * NSUB + s
        row0 = s * RPT
        ibufs = (i0, i1, i2, i3)
        isems = (semi0, semi1, semi2, semi3)
        rbufs = (bufa, bufb)
        rsems = (sema, semb)

        def idx_load(ch, p):
            pltpu.async_copy(edges_hbm.at[wid, ch], ibufs[p], isems[p])

        def idx_wait(p):
            pltpu.make_async_copy(edges_hbm.at[wid, 0], ibufs[p],
                                  isems[p]).wait()

        def gather(p4, p2):
            pltpu.async_copy(hp_hbm.at[ibufs[p4].at[0]], rbufs[p2], rsems[p2])

        def gather_wait(p2):
            pltpu.make_async_copy(hp_hbm.at[ibufs[0].at[0]], rbufs[p2],
                                  rsems[p2]).wait()

        def scatter(p4, p2):
            pltpu.sync_copy(rbufs[p2], acc.at[ibufs[p4].at[1]], add=True)

        pltpu.sync_copy(zero_hbm, acc.at[pl.ds(row0, RPT)])
        plsc.subcore_barrier()

        pltpu.sync_copy(edges_hbm.at[wid, 0], i0)
        pltpu.sync_copy(edges_hbm.at[wid, 1], i1)
        gather(0, 0)
        gather(1, 1)
        idx_load(2, 2)
        idx_load(3, 3)

        def body(m, carry):
            q = 4 * m
            for p in range(4):
                p2 = p & 1
                gather_wait(p2)
                scatter(p, p2)
                idx_load(q + p + 4, p)
                idx_wait((p + 2) & 3)
                gather((p + 2) & 3, p2)
            return carry

        lax.fori_loop(0, NCH_ // 4 - 1, body, 0)
        gather_wait(0)
        scatter(0, 0)
        idx_wait(2)
        gather(2, 0)
        gather_wait(1)
        scatter(1, 1)
        idx_wait(3)
        gather(3, 1)
        gather_wait(0)
        scatter(2, 0)
        gather_wait(1)
        scatter(3, 1)

        plsc.subcore_barrier()
        pltpu.sync_copy(acc.at[pl.ds(row0, RPT)],
                        out_hbm.at[c, pl.ds(row0, RPT)])

    return k(hp, edges3, zero_blk)


def _dinv_block(p):
    deg = p[0, :, 0:1] + p[1, :, 0:1] + 1.0
    return lax.rsqrt(deg)


def _tc_first(x, W, P):
    def body(x_ref, w_ref, p_ref, o_ref):
        dinv = _dinv_block(p_ref[...])
        o_ref[...] = dinv * jnp.dot(x_ref[...], w_ref[...],
                                    preferred_element_type=jnp.float32)

    return pl.pallas_call(
        body,
        grid=(N // BM,),
        in_specs=[
            pl.BlockSpec((BM, D), lambda i: (i, 0)),
            pl.BlockSpec((D, D), lambda i: (0, 0)),
            pl.BlockSpec((NCORES, BM, 16), lambda i: (0, i, 0)),
        ],
        out_specs=pl.BlockSpec((BM, D), lambda i: (i, 0)),
        out_shape=jax.ShapeDtypeStruct((N, D), jnp.float32),
    )(x, W, P)


def _tc_mid(A, hp, P, b, W):
    def body(a_ref, hp_ref, p_ref, b_ref, w_ref, o_ref):
        dinv = _dinv_block(p_ref[...])
        a = a_ref[...]
        z = dinv * (a[0] + a[1] + hp_ref[...]) + b_ref[...]
        h = jnp.maximum(z, 0.0)
        o_ref[...] = dinv * jnp.dot(h, w_ref[...],
                                    preferred_element_type=jnp.float32)

    return pl.pallas_call(
        body,
        grid=(N // BM,),
        in_specs=[
            pl.BlockSpec((NCORES, BM, D), lambda i: (0, i, 0)),
            pl.BlockSpec((BM, D), lambda i: (i, 0)),
            pl.BlockSpec((NCORES, BM, 16), lambda i: (0, i, 0)),
            pl.BlockSpec((1, D), lambda i: (0, 0)),
            pl.BlockSpec((D, D), lambda i: (0, 0)),
        ],
        out_specs=pl.BlockSpec((BM, D), lambda i: (i, 0)),
        out_shape=jax.ShapeDtypeStruct((N, D), jnp.float32),
    )(A, hp, P, b, W)


def _tc_last(A, hp, P, b):
    def body(a_ref, hp_ref, p_ref, b_ref, o_ref):
        dinv = _dinv_block(p_ref[...])
        a = a_ref[...]
        z = dinv * (a[0] + a[1] + hp_ref[...]) + b_ref[...]
        m = jnp.max(z, axis=1, keepdims=True)
        ez = jnp.exp(z - m)
        o_ref[...] = (z - m) - jnp.log(jnp.sum(ez, axis=1, keepdims=True))

    return pl.pallas_call(
        body,
        grid=(N // BM,),
        in_specs=[
            pl.BlockSpec((NCORES, BM, D), lambda i: (0, i, 0)),
            pl.BlockSpec((BM, D), lambda i: (i, 0)),
            pl.BlockSpec((NCORES, BM, 16), lambda i: (0, i, 0)),
            pl.BlockSpec((1, D), lambda i: (0, 0)),
        ],
        out_specs=pl.BlockSpec((BM, D), lambda i: (i, 0)),
        out_shape=jax.ShapeDtypeStruct((N, D), jnp.float32),
    )(A, hp, P, b)


def kernel(x, edge_index, W1, b1, W2, b2, W3, b3):
    ei32 = edge_index.astype(jnp.int32)
    edges3 = ei32.reshape(2, NTILES, NCH, K).transpose(1, 2, 0, 3)
    edges_deg = ei32.reshape(2, NTILES, 16, EPT // 16).transpose(1, 2, 0, 3)
    zero_d = jnp.zeros((RPT, D), jnp.float32)
    zero16 = jnp.zeros((RPT, 16), jnp.float32)
    ones16 = jnp.ones((N, 16), jnp.float32)
    b1r = b1.reshape(1, D)
    b2r = b2.reshape(1, D)
    b3r = b3.reshape(1, D)

    P = _sc_agg(ones16, edges_deg, zero16)
    hp1 = _tc_first(x, W1, P)
    A1 = _sc_agg(hp1, edges3, zero_d)
    hp2 = _tc_mid(A1, hp1, P, b1r, W2)
    A2 = _sc_agg(hp2, edges3, zero_d)
    hp3 = _tc_mid(A2, hp2, P, b2r, W3)
    A3 = _sc_agg(hp3, edges3, zero_d)
    return _tc_last(A3, hp3, P, b3r)


# speedup vs baseline: 29.4995x; 1.0619x over previous
import functools

import jax
import jax.numpy as jnp
from jax import lax
from jax.experimental import pallas as pl
from jax.experimental.pallas import tpu as pltpu
from jax.experimental.pallas import tpu_sc as plsc

N = 10000
NPAD = 10240
E = 320000
D = 128
NCORES = 2
NSUB = 16
NTILES = NCORES * NSUB
EPT = E // NTILES
K = 125
NCH = EPT // K
RPT = NPAD // NSUB
BM = 1000


def _sc_mesh():
    return plsc.VectorSubcoreMesh(core_axis_name="c", subcore_axis_name="s")


def _sc_agg(hp, edges3, zero_blk):
    W = hp.shape[1]
    dt = hp.dtype
    NCH_, K_ = edges3.shape[1], edges3.shape[3]

    @functools.partial(
        pl.kernel,
        mesh=_sc_mesh(),
        compiler_params=pltpu.CompilerParams(
            use_tc_tiling_on_sc=(W == D and dt == jnp.float32)),
        out_type=jax.ShapeDtypeStruct((NCORES, NPAD, W), dt),
        scratch_types=[
            pltpu.VMEM((2, K_), jnp.int32),
            pltpu.VMEM((2, K_), jnp.int32),
            pltpu.VMEM((2, K_), jnp.int32),
            pltpu.VMEM((2, K_), jnp.int32),
            pltpu.VMEM((K_, W), dt),
            pltpu.VMEM((K_, W), dt),
            pltpu.VMEM_SHARED((NPAD, W), dt),
            pltpu.SemaphoreType.DMA,
            pltpu.SemaphoreType.DMA,
            pltpu.SemaphoreType.DMA,
            pltpu.SemaphoreType.DMA,
            pltpu.SemaphoreType.DMA,
            pltpu.SemaphoreType.DMA,
        ],
    )
    def k(hp_hbm, edges_hbm, zero_hbm, out_hbm,
          i0, i1, i2, i3, bufa, bufb, acc,
          sema, semb, semi0, semi1, semi2, semi3):
        c = lax.axis_index("c")
        s = lax.axis_index("s")
        wid = c * NSUB + s
        row0 = s * RPT
        ibufs = (i0, i1, i2, i3)
        isems = (semi0, semi1, semi2, semi3)
        rbufs = (bufa, bufb)
        rsems = (sema, semb)

        def idx_load(ch, p):
            pltpu.async_copy(edges_hbm.at[wid, ch], ibufs[p], isems[p])

        def idx_wait(p):
            pltpu.make_async_copy(edges_hbm.at[wid, 0], ibufs[p],
                                  isems[p]).wait()

        def gather(p4, p2):
            pltpu.async_copy(hp_hbm.at[ibufs[p4].at[0]], rbufs[p2], rsems[p2])

        def gather_wait(p2):
            pltpu.make_async_copy(hp_hbm.at[ibufs[0].at[0]], rbufs[p2],
                                  rsems[p2]).wait()

        def scatter(p4, p2):
            pltpu.sync_copy(rbufs[p2], acc.at[ibufs[p4].at[1]], add=True)

        pltpu.sync_copy(zero_hbm, acc.at[pl.ds(row0, RPT)])
        plsc.subcore_barrier()

        pltpu.sync_copy(edges_hbm.at[wid, 0], i0)
        pltpu.sync_copy(edges_hbm.at[wid, 1], i1)
        gather(0, 0)
        gather(1, 1)
        idx_load(2, 2)
        idx_load(3, 3)

        def body(m, carry):
            q = 4 * m
            for p in range(4):
                p2 = p & 1
                gather_wait(p2)
                scatter(p, p2)
                idx_load(q + p + 4, p)
                idx_wait((p + 2) & 3)
                gather((p + 2) & 3, p2)
            return carry

        lax.fori_loop(0, NCH_ // 4 - 1, body, 0)
        gather_wait(0)
        scatter(0, 0)
        idx_wait(2)
        gather(2, 0)
        gather_wait(1)
        scatter(1, 1)
        idx_wait(3)
        gather(3, 1)
        gather_wait(0)
        scatter(2, 0)
        gather_wait(1)
        scatter(3, 1)

        plsc.subcore_barrier()
        pltpu.sync_copy(acc.at[pl.ds(row0, RPT)],
                        out_hbm.at[c, pl.ds(row0, RPT)])

    return k(hp, edges3, zero_blk)


def _dinv_block(p):
    deg = p[0, :, 0:1] + p[1, :, 0:1] + 1.0
    return lax.rsqrt(deg)


def _tc_first(x, W, P):
    def body(x_ref, w_ref, p_ref, o_ref):
        dinv = _dinv_block(p_ref[...])
        o_ref[...] = (dinv * jnp.dot(x_ref[...], w_ref[...],
                                     preferred_element_type=jnp.float32)
                      ).astype(jnp.bfloat16)

    return pl.pallas_call(
        body,
        grid=(N // BM,),
        in_specs=[
            pl.BlockSpec((BM, D), lambda i: (i, 0)),
            pl.BlockSpec((D, D), lambda i: (0, 0)),
            pl.BlockSpec((NCORES, BM, 16), lambda i: (0, i, 0)),
        ],
        out_specs=pl.BlockSpec((BM, D), lambda i: (i, 0)),
        out_shape=jax.ShapeDtypeStruct((N, D), jnp.bfloat16),
    )(x, W, P)


def _tc_mid(A, hp, P, b, W):
    def body(a_ref, hp_ref, p_ref, b_ref, w_ref, o_ref):
        dinv = _dinv_block(p_ref[...])
        a = a_ref[...].astype(jnp.float32)
        hpv = hp_ref[...].astype(jnp.float32)
        z = dinv * (a[0] + a[1] + hpv) + b_ref[...]
        h = jnp.maximum(z, 0.0)
        o_ref[...] = (dinv * jnp.dot(h, w_ref[...],
                                     preferred_element_type=jnp.float32)
                      ).astype(jnp.bfloat16)

    return pl.pallas_call(
        body,
        grid=(N // BM,),
        in_specs=[
            pl.BlockSpec((NCORES, BM, D), lambda i: (0, i, 0)),
            pl.BlockSpec((BM, D), lambda i: (i, 0)),
            pl.BlockSpec((NCORES, BM, 16), lambda i: (0, i, 0)),
            pl.BlockSpec((1, D), lambda i: (0, 0)),
            pl.BlockSpec((D, D), lambda i: (0, 0)),
        ],
        out_specs=pl.BlockSpec((BM, D), lambda i: (i, 0)),
        out_shape=jax.ShapeDtypeStruct((N, D), jnp.bfloat16),
    )(A, hp, P, b, W)


def _tc_last(A, hp, P, b):
    def body(a_ref, hp_ref, p_ref, b_ref, o_ref):
        dinv = _dinv_block(p_ref[...])
        a = a_ref[...].astype(jnp.float32)
        hpv = hp_ref[...].astype(jnp.float32)
        z = dinv * (a[0] + a[1] + hpv) + b_ref[...]
        m = jnp.max(z, axis=1, keepdims=True)
        ez = jnp.exp(z - m)
        o_ref[...] = (z - m) - jnp.log(jnp.sum(ez, axis=1, keepdims=True))

    return pl.pallas_call(
        body,
        grid=(N // BM,),
        in_specs=[
            pl.BlockSpec((NCORES, BM, D), lambda i: (0, i, 0)),
            pl.BlockSpec((BM, D), lambda i: (i, 0)),
            pl.BlockSpec((NCORES, BM, 16), lambda i: (0, i, 0)),
            pl.BlockSpec((1, D), lambda i: (0, 0)),
        ],
        out_specs=pl.BlockSpec((BM, D), lambda i: (i, 0)),
        out_shape=jax.ShapeDtypeStruct((N, D), jnp.float32),
    )(A, hp, P, b)


def kernel(x, edge_index, W1, b1, W2, b2, W3, b3):
    ei32 = edge_index.astype(jnp.int32)
    edges3 = ei32.reshape(2, NTILES, NCH, K).transpose(1, 2, 0, 3)
    edges_deg = ei32.reshape(2, NTILES, 16, EPT // 16).transpose(1, 2, 0, 3)
    zero_d = jnp.zeros((RPT, D), jnp.bfloat16)
    zero16 = jnp.zeros((RPT, 16), jnp.float32)
    ones16 = jnp.ones((N, 16), jnp.float32)
    b1r = b1.reshape(1, D)
    b2r = b2.reshape(1, D)
    b3r = b3.reshape(1, D)

    P = _sc_agg(ones16, edges_deg, zero16)
    hp1 = _tc_first(x, W1, P)
    A1 = _sc_agg(hp1, edges3, zero_d)
    hp2 = _tc_mid(A1, hp1, P, b1r, W2)
    A2 = _sc_agg(hp2, edges3, zero_d)
    hp3 = _tc_mid(A2, hp2, P, b2r, W3)
    A3 = _sc_agg(hp3, edges3, zero_d)
    return _tc_last(A3, hp3, P, b3r)


# speedup vs baseline: 30.1772x; 1.0230x over previous
import functools

import jax
import jax.numpy as jnp
from jax import lax
from jax.experimental import pallas as pl
from jax.experimental.pallas import tpu as pltpu
from jax.experimental.pallas import tpu_sc as plsc

N = 10000
NPAD = 10240
E = 320000
D = 128
NCORES = 2
NSUB = 16
NTILES = NCORES * NSUB
EPT = E // NTILES
K = 125
NCH = EPT // K
RPT = NPAD // NSUB
BM = 2000


def _sc_mesh():
    return plsc.VectorSubcoreMesh(core_axis_name="c", subcore_axis_name="s")


def _sc_agg(hp, edges3, zero_blk):
    W = hp.shape[1]
    dt = hp.dtype
    NCH_, K_ = edges3.shape[1], edges3.shape[3]

    @functools.partial(
        pl.kernel,
        mesh=_sc_mesh(),
        compiler_params=pltpu.CompilerParams(
            use_tc_tiling_on_sc=(W == D and dt == jnp.float32)),
        out_type=jax.ShapeDtypeStruct((NCORES, NPAD, W), dt),
        scratch_types=[
            pltpu.VMEM((2, K_), jnp.int32),
            pltpu.VMEM((2, K_), jnp.int32),
            pltpu.VMEM((2, K_), jnp.int32),
            pltpu.VMEM((2, K_), jnp.int32),
            pltpu.VMEM((K_, W), dt),
            pltpu.VMEM((K_, W), dt),
            pltpu.VMEM_SHARED((NPAD, W), dt),
            pltpu.SemaphoreType.DMA,
            pltpu.SemaphoreType.DMA,
            pltpu.SemaphoreType.DMA,
            pltpu.SemaphoreType.DMA,
            pltpu.SemaphoreType.DMA,
            pltpu.SemaphoreType.DMA,
        ],
    )
    def k(hp_hbm, edges_hbm, zero_hbm, out_hbm,
          i0, i1, i2, i3, bufa, bufb, acc,
          sema, semb, semi0, semi1, semi2, semi3):
        c = lax.axis_index("c")
        s = lax.axis_index("s")
        wid = c * NSUB + s
        row0 = s * RPT
        ibufs = (i0, i1, i2, i3)
        isems = (semi0, semi1, semi2, semi3)
        rbufs = (bufa, bufb)
        rsems = (sema, semb)

        def idx_load(ch, p):
            pltpu.async_copy(edges_hbm.at[wid, ch], ibufs[p], isems[p])

        def idx_wait(p):
            pltpu.make_async_copy(edges_hbm.at[wid, 0], ibufs[p],
                                  isems[p]).wait()

        def gather(p4, p2):
            pltpu.async_copy(hp_hbm.at[ibufs[p4].at[0]], rbufs[p2], rsems[p2])

        def gather_wait(p2):
            pltpu.make_async_copy(hp_hbm.at[ibufs[0].at[0]], rbufs[p2],
                                  rsems[p2]).wait()

        def scatter(p4, p2):
            pltpu.sync_copy(rbufs[p2], acc.at[ibufs[p4].at[1]], add=True)

        pltpu.sync_copy(zero_hbm, acc.at[pl.ds(row0, RPT)])
        plsc.subcore_barrier()

        pltpu.sync_copy(edges_hbm.at[wid, 0], i0)
        pltpu.sync_copy(edges_hbm.at[wid, 1], i1)
        gather(0, 0)
        gather(1, 1)
        idx_load(2, 2)
        idx_load(3, 3)

        def body(m, carry):
            q = 4 * m
            for p in range(4):
                p2 = p & 1
                gather_wait(p2)
                scatter(p, p2)
                idx_load(q + p + 4, p)
                idx_wait((p + 2) & 3)
                gather((p + 2) & 3, p2)
            return carry

        lax.fori_loop(0, NCH_ // 4 - 1, body, 0)
        gather_wait(0)
        scatter(0, 0)
        idx_wait(2)
        gather(2, 0)
        gather_wait(1)
        scatter(1, 1)
        idx_wait(3)
        gather(3, 1)
        gather_wait(0)
        scatter(2, 0)
        gather_wait(1)
        scatter(3, 1)

        plsc.subcore_barrier()
        pltpu.sync_copy(acc.at[pl.ds(row0, RPT)],
                        out_hbm.at[c, pl.ds(row0, RPT)])

    return k(hp, edges3, zero_blk)


def _dinv_block(p):
    deg = p[0, :, 0:1] + p[1, :, 0:1] + 1.0
    return lax.rsqrt(deg)


def _tc_first(x, W, P):
    def body(x_ref, w_ref, p_ref, o_ref):
        dinv = _dinv_block(p_ref[...])
        o_ref[...] = (dinv * jnp.dot(x_ref[...], w_ref[...],
                                     preferred_element_type=jnp.float32)
                      ).astype(jnp.bfloat16)

    return pl.pallas_call(
        body,
        grid=(N // BM,),
        in_specs=[
            pl.BlockSpec((BM, D), lambda i: (i, 0)),
            pl.BlockSpec((D, D), lambda i: (0, 0)),
            pl.BlockSpec((NCORES, BM, 16), lambda i: (0, i, 0)),
        ],
        out_specs=pl.BlockSpec((BM, D), lambda i: (i, 0)),
        out_shape=jax.ShapeDtypeStruct((N, D), jnp.bfloat16),
    )(x, W, P)


def _tc_mid(A, hp, P, b, W):
    def body(a_ref, hp_ref, p_ref, b_ref, w_ref, o_ref):
        dinv = _dinv_block(p_ref[...])
        a = a_ref[...].astype(jnp.float32)
        hpv = hp_ref[...].astype(jnp.float32)
        z = dinv * (a[0] + a[1] + hpv) + b_ref[...]
        h = jnp.maximum(z, 0.0)
        o_ref[...] = (dinv * jnp.dot(h, w_ref[...],
                                     preferred_element_type=jnp.float32)
                      ).astype(jnp.bfloat16)

    return pl.pallas_call(
        body,
        grid=(N // BM,),
        in_specs=[
            pl.BlockSpec((NCORES, BM, D), lambda i: (0, i, 0)),
            pl.BlockSpec((BM, D), lambda i: (i, 0)),
            pl.BlockSpec((NCORES, BM, 16), lambda i: (0, i, 0)),
            pl.BlockSpec((1, D), lambda i: (0, 0)),
            pl.BlockSpec((D, D), lambda i: (0, 0)),
        ],
        out_specs=pl.BlockSpec((BM, D), lambda i: (i, 0)),
        out_shape=jax.ShapeDtypeStruct((N, D), jnp.bfloat16),
    )(A, hp, P, b, W)


def _tc_last(A, hp, P, b):
    def body(a_ref, hp_ref, p_ref, b_ref, o_ref):
        dinv = _dinv_block(p_ref[...])
        a = a_ref[...].astype(jnp.float32)
        hpv = hp_ref[...].astype(jnp.float32)
        z = dinv * (a[0] + a[1] + hpv) + b_ref[...]
        m = jnp.max(z, axis=1, keepdims=True)
        ez = jnp.exp(z - m)
        o_ref[...] = (z - m) - jnp.log(jnp.sum(ez, axis=1, keepdims=True))

    return pl.pallas_call(
        body,
        grid=(N // BM,),
        in_specs=[
            pl.BlockSpec((NCORES, BM, D), lambda i: (0, i, 0)),
            pl.BlockSpec((BM, D), lambda i: (i, 0)),
            pl.BlockSpec((NCORES, BM, 16), lambda i: (0, i, 0)),
            pl.BlockSpec((1, D), lambda i: (0, 0)),
        ],
        out_specs=pl.BlockSpec((BM, D), lambda i: (i, 0)),
        out_shape=jax.ShapeDtypeStruct((N, D), jnp.float32),
    )(A, hp, P, b)


def kernel(x, edge_index, W1, b1, W2, b2, W3, b3):
    ei32 = edge_index.astype(jnp.int32)
    edges3 = ei32.reshape(2, NTILES, NCH, K).transpose(1, 2, 0, 3)
    edges_deg = ei32.reshape(2, NTILES, 16, EPT // 16).transpose(1, 2, 0, 3)
    zero_d = jnp.zeros((RPT, D), jnp.bfloat16)
    zero16 = jnp.zeros((RPT, 16), jnp.float32)
    ones16 = jnp.ones((N, 16), jnp.float32)
    b1r = b1.reshape(1, D)
    b2r = b2.reshape(1, D)
    b3r = b3.reshape(1, D)

    P = _sc_agg(ones16, edges_deg, zero16)
    hp1 = _tc_first(x, W1, P)
    A1 = _sc_agg(hp1, edges3, zero_d)
    hp2 = _tc_mid(A1, hp1, P, b1r, W2)
    A2 = _sc_agg(hp2, edges3, zero_d)
    hp3 = _tc_mid(A2, hp2, P, b2r, W3)
    A3 = _sc_agg(hp3, edges3, zero_d)
    return _tc_last(A3, hp3, P, b3r)
